# baseline (speedup 1.0000x reference)
import functools
from itertools import accumulate

import jax
import jax.numpy as jnp
from jax import lax
from jax.experimental import pallas as pl
from jax.experimental.pallas import tpu as pltpu
from jax.experimental.pallas import tpu_sc as plsc

_FIELD_DIMS = [40000] * 26
_EMBED = 64
_BATCH = 4096
_NF = len(_FIELD_DIMS)

_NC = 2
_NS = 16
_NW = _NC * _NS

_B_TOTAL = _BATCH * _NF
_B_PER_W = _B_TOTAL // _NW
_CH = 128
_NCH = _B_PER_W // _CH


def _make_sc_gather():
    mesh = plsc.VectorSubcoreMesh(core_axis_name="c", subcore_axis_name="s")

    @functools.partial(
        pl.kernel,
        mesh=mesh,
        compiler_params=pltpu.CompilerParams(use_tc_tiling_on_sc=False),
        out_type=jax.ShapeDtypeStruct((_B_TOTAL, _EMBED), jnp.float32),
        scratch_types=[
            pltpu.VMEM((_B_PER_W,), jnp.int32),
            pltpu.VMEM((_B_PER_W,), jnp.int32),
            pltpu.VMEM((_CH, _EMBED), jnp.float32),
            pltpu.SemaphoreType.DMA,
        ],
    )
    def k(idx_hbm, off_hbm, w_hbm, out_hbm, idx_v, off_v, rows_v, gsem):
        wid = lax.axis_index("s") * _NC + lax.axis_index("c")
        base = wid * _B_PER_W

        pltpu.sync_copy(idx_hbm.at[pl.ds(base, _B_PER_W)], idx_v)
        pltpu.sync_copy(off_hbm, off_v)

        def add_body(i, carry):
            c = i * 16
            idx_v[pl.ds(c, 16)] = idx_v[pl.ds(c, 16)] + off_v[pl.ds(c, 16)]
            return carry

        lax.fori_loop(0, _B_PER_W // 16, add_body, 0)

        def gather_body(s, carry):
            pltpu.async_copy(
                w_hbm.at[idx_v.at[pl.ds(s * _CH, _CH)]], rows_v, gsem
            ).wait()
            pltpu.sync_copy(rows_v, out_hbm.at[pl.ds(base + s * _CH, _CH)])
            return carry

        lax.fori_loop(0, _NCH, gather_body, 0)

    return k


_sc_gather = _make_sc_gather()


def kernel(x, weight):
    offset = jnp.asarray(
        [0, *accumulate(_FIELD_DIMS)][:-1], dtype=jnp.int32
    )
    off_tile = jnp.tile(offset, _B_PER_W // _NF)
    idx_flat = x.astype(jnp.int32).reshape(_B_TOTAL)
    out = _sc_gather(idx_flat, off_tile, weight)
    return out.reshape(_BATCH, _NF, _EMBED)


# speedup vs baseline: 1.0273x; 1.0273x over previous
import functools
from itertools import accumulate

import jax
import jax.numpy as jnp
from jax import lax
from jax.experimental import pallas as pl
from jax.experimental.pallas import tpu as pltpu
from jax.experimental.pallas import tpu_sc as plsc

_FIELD_DIMS = [40000] * 26
_EMBED = 64
_BATCH = 4096
_NF = len(_FIELD_DIMS)

_NC = 2
_NS = 16
_NW = _NC * _NS

_B_TOTAL = _BATCH * _NF
_B_PER_W = _B_TOTAL // _NW
_CH = 256
_NCH = _B_PER_W // _CH
_IDXROWS = _B_PER_W // 128
_NBUF = 4
_LOOKAHEAD = 2


def _make_sc_gather():
    mesh = plsc.VectorSubcoreMesh(core_axis_name="c", subcore_axis_name="s")

    @functools.partial(
        pl.kernel,
        mesh=mesh,
        compiler_params=pltpu.CompilerParams(use_tc_tiling_on_sc=False),
        out_type=jax.ShapeDtypeStruct((_B_TOTAL, _EMBED), jnp.float32),
        scratch_types=[
            pltpu.VMEM((_B_PER_W,), jnp.int32),
            pltpu.VMEM((_B_PER_W,), jnp.int32),
            pltpu.VMEM((_NBUF, _CH, _EMBED), jnp.float32),
            pltpu.SemaphoreType.DMA,
            pltpu.SemaphoreType.DMA,
        ],
    )
    def k(idx_hbm, off_hbm, w_hbm, out_hbm, idx_v, off_v, rows_v, gsem, osem):
        wid = lax.axis_index("s") * _NC + lax.axis_index("c")
        base = wid * _B_PER_W

        pltpu.sync_copy(idx_hbm.at[pl.ds(base, _B_PER_W)], idx_v)
        pltpu.sync_copy(off_hbm, off_v)

        def add_body(i, carry):
            c = i * 16
            idx_v[pl.ds(c, 16)] = idx_v[pl.ds(c, 16)] + off_v[pl.ds(c, 16)]
            return carry

        lax.fori_loop(0, _B_PER_W // 16, add_body, 0)

        def start_gather(t):
            return pltpu.async_copy(
                w_hbm.at[idx_v.at[pl.ds(t * _CH, _CH)]],
                rows_v.at[t % _NBUF],
                gsem,
            )

        h_g = [None] * _NCH
        h_o = [None] * _NCH
        for t in range(_LOOKAHEAD):
            h_g[t] = start_gather(t)
        for s in range(_NCH):
            t = s + _LOOKAHEAD
            if t < _NCH:
                if t - _NBUF >= 0:
                    h_o[t - _NBUF].wait()
                h_g[t] = start_gather(t)
            h_g[s].wait()
            h_o[s] = pltpu.async_copy(
                rows_v.at[s % _NBUF],
                out_hbm.at[pl.ds(base + s * _CH, _CH)],
                osem,
            )
        for s in range(max(0, _NCH - _NBUF), _NCH):
            h_o[s].wait()

    return k


_sc_gather = _make_sc_gather()


def kernel(x, weight):
    offset = jnp.asarray(
        [0, *accumulate(_FIELD_DIMS)][:-1], dtype=jnp.int32
    )
    off_tile = jnp.tile(offset, _B_PER_W // _NF)
    idx_flat = x.astype(jnp.int32).reshape(_B_TOTAL)
    out = _sc_gather(idx_flat, off_tile, weight)
    return out.reshape(_BATCH, _NF, _EMBED)


# speedup vs baseline: 1.0277x; 1.0004x over previous
import functools
from itertools import accumulate

import jax
import jax.numpy as jnp
from jax import lax
from jax.experimental import pallas as pl
from jax.experimental.pallas import tpu as pltpu
from jax.experimental.pallas import tpu_sc as plsc

_FIELD_DIMS = [40000] * 26
_EMBED = 64
_BATCH = 4096
_NF = len(_FIELD_DIMS)

_NC = 2
_NS = 16
_NW = _NC * _NS

_B_TOTAL = _BATCH * _NF
_B_PER_W = _B_TOTAL // _NW
_CH = 256
_NCH = _B_PER_W // _CH
_NBUF = 4
_LOOKAHEAD = 2


def _make_sc_gather():
    mesh = plsc.VectorSubcoreMesh(core_axis_name="c", subcore_axis_name="s")

    @functools.partial(
        pl.kernel,
        mesh=mesh,
        compiler_params=pltpu.CompilerParams(use_tc_tiling_on_sc=False),
        out_type=jax.ShapeDtypeStruct((_B_TOTAL, _EMBED), jnp.float32),
        scratch_types=[
            pltpu.VMEM((_B_PER_W,), jnp.int32),
            pltpu.VMEM((_NBUF, _CH, _EMBED), jnp.float32),
            pltpu.SemaphoreType.DMA,
            pltpu.SemaphoreType.DMA,
        ],
    )
    def k(idx_hbm, w_hbm, out_hbm, idx_v, rows_v, gsem, osem):
        wid = lax.axis_index("s") * _NC + lax.axis_index("c")
        base = wid * _B_PER_W

        pltpu.sync_copy(idx_hbm.at[pl.ds(base, _B_PER_W)], idx_v)

        def start_gather(t):
            return pltpu.async_copy(
                w_hbm.at[idx_v.at[pl.ds(t * _CH, _CH)]],
                rows_v.at[t % _NBUF],
                gsem,
            )

        h_g = [None] * _NCH
        h_o = [None] * _NCH
        for t in range(_LOOKAHEAD):
            h_g[t] = start_gather(t)
        for s in range(_NCH):
            t = s + _LOOKAHEAD
            if t < _NCH:
                if t - _NBUF >= 0:
                    h_o[t - _NBUF].wait()
                h_g[t] = start_gather(t)
            h_g[s].wait()
            h_o[s] = pltpu.async_copy(
                rows_v.at[s % _NBUF],
                out_hbm.at[pl.ds(base + s * _CH, _CH)],
                osem,
            )
        for s in range(max(0, _NCH - _NBUF), _NCH):
            h_o[s].wait()

    return k


_sc_gather = _make_sc_gather()

def kernel(x, weight):
    offset = jnp.asarray(
        [0, *accumulate(_FIELD_DIMS)][:-1], dtype=jnp.int32
    )
    idx_flat = (x.astype(jnp.int32) + offset[None, :]).reshape(_B_TOTAL)
    inter = _sc_gather(idx_flat, weight)
    return inter.reshape(_BATCH, _NF, _EMBED)


# speedup vs baseline: 1.1156x; 1.0855x over previous
import functools
from itertools import accumulate

import jax
import jax.numpy as jnp
from jax import lax
from jax.experimental import pallas as pl
from jax.experimental.pallas import tpu as pltpu
from jax.experimental.pallas import tpu_sc as plsc

_FIELD_DIMS = [40000] * 26
_EMBED = 64
_BATCH = 4096
_NF = len(_FIELD_DIMS)

_NC = 2
_NS = 16
_NW = _NC * _NS

_B_TOTAL = _BATCH * _NF
_B_PER_W = _B_TOTAL // _NW
_CH = 128
_NCH = _B_PER_W // _CH
_NBUF = 4
_LOOKAHEAD = 2

_LANES = 128


def _make_sc_gather():
    mesh = plsc.VectorSubcoreMesh(core_axis_name="c", subcore_axis_name="s")

    @functools.partial(
        pl.kernel,
        mesh=mesh,
        compiler_params=pltpu.CompilerParams(use_tc_tiling_on_sc=False),
        out_type=jax.ShapeDtypeStruct((_B_TOTAL, _EMBED), jnp.float32),
        scratch_types=[
            pltpu.VMEM((_B_PER_W,), jnp.int32),
            pltpu.VMEM((_NBUF, _CH, _LANES), jnp.float32),
            pltpu.SemaphoreType.DMA,
            pltpu.SemaphoreType.DMA,
        ],
    )
    def k(idx_hbm, w_hbm, out_hbm, idx_v, rows_v, gsem, osem):
        wid = lax.axis_index("s") * _NC + lax.axis_index("c")
        base = wid * _B_PER_W

        pltpu.sync_copy(idx_hbm.at[pl.ds(base, _B_PER_W)], idx_v)

        def start_gather(t):
            return pltpu.async_copy(
                w_hbm.at[idx_v.at[pl.ds(t * _CH, _CH)]],
                rows_v.at[t % _NBUF],
                gsem,
            )

        h_g = [None] * _NCH
        h_o = [None] * _NCH
        for t in range(_LOOKAHEAD):
            h_g[t] = start_gather(t)
        for s in range(_NCH):
            t = s + _LOOKAHEAD
            if t < _NCH:
                if t - _NBUF >= 0:
                    h_o[t - _NBUF].wait()
                h_g[t] = start_gather(t)
            h_g[s].wait()
            h_o[s] = pltpu.async_copy(
                rows_v.at[s % _NBUF, pl.ds(0, _CH), pl.ds(0, _EMBED)],
                out_hbm.at[pl.ds(base + s * _CH, _CH)],
                osem,
            )
        for s in range(max(0, _NCH - _NBUF), _NCH):
            h_o[s].wait()

    return k


_sc_gather = _make_sc_gather()


def kernel(x, weight):
    offset = jnp.asarray(
        [0, *accumulate(_FIELD_DIMS)][:-1], dtype=jnp.int32
    )
    idx_flat = (x.astype(jnp.int32) + offset[None, :]).reshape(_B_TOTAL)
    wp = jnp.pad(weight, ((0, 0), (0, _LANES - _EMBED)))
    inter = _sc_gather(idx_flat, wp)
    return inter.reshape(_BATCH, _NF, _EMBED)
